# initial kernel scaffold (unmeasured)
import jax
import jax.numpy as jnp
from jax import lax
from jax.experimental import pallas as pl
from jax.experimental.pallas import tpu as pltpu

N_DEV = 4
B, SQ, SKV, HQ, DH = 2, 128, 512, 16, 64
H_LOC = HQ // N_DEV
KV_SRC = 2
SKV_LOC = SKV // N_DEV
SKV_EFF = KV_SRC * SKV_LOC
D_MODEL = 512
WINDOW = 128
NEG = -1e9


def kernel(x, Wq, K_ext, V_ext, Wo):
    def body(x_ref, wq_ref, k_ref, v_ref, wo_ref, out_ref,
             kbuf, vbuf, pbuf,
             kv_send_sems, kv_recv_sems, loc_sems,
             ar_send_sems, ar_recv_sems):
        my = lax.axis_index("i")

        barrier = pltpu.get_barrier_semaphore()
        for off in range(1, N_DEV):
            pl.semaphore_signal(
                barrier, inc=1,
                device_id=((my + off) % N_DEV,),
                device_id_type=pl.DeviceIdType.MESH,
            )
        pl.semaphore_wait(barrier, N_DEV - 1)

        for s in range(KV_SRC):
            @pl.when(my == s)
            def _(s=s):
                for d in range(N_DEV):
                    if d == s:
                        continue
                    for t, (src, dst) in enumerate(((k_ref, kbuf), (v_ref, vbuf))):
                        pltpu.make_async_remote_copy(
                            src_ref=src.at[:, :, pl.ds(d * H_LOC, H_LOC), :],
                            dst_ref=dst.at[s],
                            send_sem=kv_send_sems.at[d, t],
                            recv_sem=kv_recv_sems.at[s, t],
                            device_id=(d,),
                            device_id_type=pl.DeviceIdType.MESH,
                        ).start()
                for t, (src, dst) in enumerate(((k_ref, kbuf), (v_ref, vbuf))):
                    cp = pltpu.make_async_copy(
                        src.at[:, :, pl.ds(s * H_LOC, H_LOC), :],
                        dst.at[s], loc_sems.at[t],
                    )
                    cp.start()
                    cp.wait()

        for s in range(KV_SRC):
            @pl.when(my != s)
            def _(s=s):
                for t, dst in ((0, kbuf), (1, vbuf)):
                    pltpu.make_async_remote_copy(
                        src_ref=k_ref.at[:, :, pl.ds(0, H_LOC), :],
                        dst_ref=dst.at[s],
                        send_sem=kv_send_sems.at[0, t],
                        recv_sem=kv_recv_sems.at[s, t],
                        device_id=(s,),
                        device_id_type=pl.DeviceIdType.MESH,
                    ).wait_recv()

        qi = lax.broadcasted_iota(jnp.int32, (SQ, SKV_EFF), 0)
        ki = lax.broadcasted_iota(jnp.int32, (SQ, SKV_EFF), 1)
        mask = ki <= qi + WINDOW

        for b in range(B):
            acc = jnp.zeros((SQ, D_MODEL), jnp.float32)
            for h in range(H_LOC):
                qh = jnp.dot(x_ref[b], wq_ref[:, h * DH:(h + 1) * DH],
                             preferred_element_type=jnp.float32)
                kc = kbuf[:, b, :, h, :].reshape(SKV_EFF, DH)
                vc = vbuf[:, b, :, h, :].reshape(SKV_EFF, DH)
                scores = lax.dot_general(
                    qh, kc, (((1,), (1,)), ((), ())),
                    preferred_element_type=jnp.float32) * 0.125
                scores = jnp.where(mask, scores, NEG)
                m = jnp.max(scores, axis=1, keepdims=True)
                w = jnp.exp(scores - m)
                w = w / jnp.sum(w, axis=1, keepdims=True)
                ctx = jnp.dot(w, vc, preferred_element_type=jnp.float32)
                acc = acc + jnp.dot(ctx, wo_ref[h * DH:(h + 1) * DH, :],
                                    preferred_element_type=jnp.float32)
            for d in range(N_DEV):
                @pl.when(my == d)
                def _(d=d, b=b, acc=acc):
                    pbuf[d, b] = acc

        for s in range(N_DEV):
            @pl.when(my == s)
            def _(s=s):
                for d in range(N_DEV):
                    if d == s:
                        continue
                    pltpu.make_async_remote_copy(
                        src_ref=pbuf.at[s], dst_ref=pbuf.at[s],
                        send_sem=ar_send_sems.at[d],
                        recv_sem=ar_recv_sems.at[s],
                        device_id=(d,),
                        device_id_type=pl.DeviceIdType.MESH,
                    ).start()

        for s in range(KV_SRC):
            @pl.when(my == s)
            def _(s=s):
                for d in range(N_DEV):
                    if d == s:
                        continue
                    for t, (src, dst) in enumerate(((k_ref, kbuf), (v_ref, vbuf))):
                        pltpu.make_async_remote_copy(
                            src_ref=src.at[:, :, pl.ds(d * H_LOC, H_LOC), :],
                            dst_ref=dst.at[s],
                            send_sem=kv_send_sems.at[d, t],
                            recv_sem=kv_recv_sems.at[s, t],
                            device_id=(d,),
                            device_id_type=pl.DeviceIdType.MESH,
                        ).wait_send()

        for s in range(N_DEV):
            @pl.when(my != s)
            def _(s=s):
                pltpu.make_async_remote_copy(
                    src_ref=pbuf.at[s], dst_ref=pbuf.at[s],
                    send_sem=ar_send_sems.at[s],
                    recv_sem=ar_recv_sems.at[s],
                    device_id=(s,),
                    device_id_type=pl.DeviceIdType.MESH,
                ).wait_recv()

        out_ref[...] = pbuf[0] + pbuf[1] + pbuf[2] + pbuf[3]

        for s in range(N_DEV):
            @pl.when(my == s)
            def _(s=s):
                for d in range(N_DEV):
                    if d == s:
                        continue
                    pltpu.make_async_remote_copy(
                        src_ref=pbuf.at[s], dst_ref=pbuf.at[s],
                        send_sem=ar_send_sems.at[d],
                        recv_sem=ar_recv_sems.at[s],
                        device_id=(d,),
                        device_id_type=pl.DeviceIdType.MESH,
                    ).wait_send()

    return pl.pallas_call(
        body,
        out_shape=jax.ShapeDtypeStruct((B, SQ, D_MODEL), jnp.float32),
        in_specs=[
            pl.BlockSpec(memory_space=pltpu.VMEM),
            pl.BlockSpec(memory_space=pltpu.VMEM),
            pl.BlockSpec(memory_space=pltpu.ANY),
            pl.BlockSpec(memory_space=pltpu.ANY),
            pl.BlockSpec(memory_space=pltpu.VMEM),
        ],
        out_specs=pl.BlockSpec(memory_space=pltpu.VMEM),
        scratch_shapes=[
            pltpu.VMEM((KV_SRC, B, SKV_LOC, H_LOC, DH), jnp.float32),
            pltpu.VMEM((KV_SRC, B, SKV_LOC, H_LOC, DH), jnp.float32),
            pltpu.VMEM((N_DEV, B, SQ, D_MODEL), jnp.float32),
            pltpu.SemaphoreType.DMA((N_DEV, 2)),
            pltpu.SemaphoreType.DMA((KV_SRC, 2)),
            pltpu.SemaphoreType.DMA((2,)),
            pltpu.SemaphoreType.DMA((N_DEV,)),
            pltpu.SemaphoreType.DMA((N_DEV,)),
        ],
        compiler_params=pltpu.CompilerParams(collective_id=0),
    )(x, Wq, K_ext, V_ext, Wo)


# baseline (device time: 47988 ns/iter reference)
import jax
import jax.numpy as jnp
from jax import lax
from jax.experimental import pallas as pl
from jax.experimental.pallas import tpu as pltpu

N_DEV = 4
B, SQ, SKV, HQ, DH = 2, 128, 512, 16, 64
H_LOC = HQ // N_DEV
KV_SRC = 2
SKV_LOC = SKV // N_DEV
SKV_EFF = KV_SRC * SKV_LOC
D_MODEL = 512
WINDOW = 128
NEG = -1e9


def kernel(x, Wq, K_ext, V_ext, Wo):
    def body(x_ref, wq_ref, k_ref, v_ref, wo_ref, out_ref,
             kbuf, vbuf, pbuf,
             kv_send_sems, kv_recv_sems, loc_sems,
             ar_send_sems, ar_recv_sems):
        my = lax.axis_index("i")

        barrier = pltpu.get_barrier_semaphore()
        for off in range(1, N_DEV):
            pl.semaphore_signal(
                barrier, inc=1,
                device_id=((my + off) % N_DEV,),
                device_id_type=pl.DeviceIdType.MESH,
            )
        pl.semaphore_wait(barrier, N_DEV - 1)

        for s in range(KV_SRC):
            @pl.when(my == s)
            def _(s=s):
                for d in range(N_DEV):
                    if d == s:
                        continue
                    for t, (src, dst) in enumerate(((k_ref, kbuf), (v_ref, vbuf))):
                        pltpu.make_async_remote_copy(
                            src_ref=src.at[:, :, pl.ds(d * H_LOC, H_LOC), :],
                            dst_ref=dst.at[s],
                            send_sem=kv_send_sems.at[d, t],
                            recv_sem=kv_recv_sems.at[s, t],
                            device_id=(d,),
                            device_id_type=pl.DeviceIdType.MESH,
                        ).start()
                for t, (src, dst) in enumerate(((k_ref, kbuf), (v_ref, vbuf))):
                    cp = pltpu.make_async_copy(
                        src.at[:, :, pl.ds(s * H_LOC, H_LOC), :],
                        dst.at[s], loc_sems.at[t],
                    )
                    cp.start()
                    cp.wait()

        for s in range(KV_SRC):
            @pl.when(my != s)
            def _(s=s):
                for t, dst in ((0, kbuf), (1, vbuf)):
                    pltpu.make_async_remote_copy(
                        src_ref=k_ref.at[:, :, pl.ds(0, H_LOC), :],
                        dst_ref=dst.at[s],
                        send_sem=kv_send_sems.at[0, t],
                        recv_sem=kv_recv_sems.at[s, t],
                        device_id=(s,),
                        device_id_type=pl.DeviceIdType.MESH,
                    ).wait_recv()

        qi = lax.broadcasted_iota(jnp.int32, (SQ, SKV_EFF), 0)
        ki = lax.broadcasted_iota(jnp.int32, (SQ, SKV_EFF), 1)
        mask = ki <= qi + WINDOW

        for b in range(B):
            acc = jnp.zeros((SQ, D_MODEL), jnp.float32)
            for h in range(H_LOC):
                qh = jnp.dot(x_ref[b], wq_ref[:, h * DH:(h + 1) * DH],
                             preferred_element_type=jnp.float32)
                kc = kbuf[:, b, :, h, :].reshape(SKV_EFF, DH)
                vc = vbuf[:, b, :, h, :].reshape(SKV_EFF, DH)
                scores = lax.dot_general(
                    qh, kc, (((1,), (1,)), ((), ())),
                    preferred_element_type=jnp.float32) * 0.125
                scores = jnp.where(mask, scores, NEG)
                m = jnp.max(scores, axis=1, keepdims=True)
                w = jnp.exp(scores - m)
                w = w / jnp.sum(w, axis=1, keepdims=True)
                ctx = jnp.dot(w, vc, preferred_element_type=jnp.float32)
                acc = acc + jnp.dot(ctx, wo_ref[h * DH:(h + 1) * DH, :],
                                    preferred_element_type=jnp.float32)
            for d in range(N_DEV):
                @pl.when(my == d)
                def _(d=d, b=b, acc=acc):
                    pbuf[d, b] = acc

        for s in range(N_DEV):
            @pl.when(my == s)
            def _(s=s):
                for d in range(N_DEV):
                    if d == s:
                        continue
                    pltpu.make_async_remote_copy(
                        src_ref=pbuf.at[s], dst_ref=pbuf.at[s],
                        send_sem=ar_send_sems.at[d],
                        recv_sem=ar_recv_sems.at[s],
                        device_id=(d,),
                        device_id_type=pl.DeviceIdType.MESH,
                    ).start()

        for s in range(KV_SRC):
            @pl.when(my == s)
            def _(s=s):
                for d in range(N_DEV):
                    if d == s:
                        continue
                    for t, (src, dst) in enumerate(((k_ref, kbuf), (v_ref, vbuf))):
                        pltpu.make_async_remote_copy(
                            src_ref=src.at[:, :, pl.ds(d * H_LOC, H_LOC), :],
                            dst_ref=dst.at[s],
                            send_sem=kv_send_sems.at[d, t],
                            recv_sem=kv_recv_sems.at[s, t],
                            device_id=(d,),
                            device_id_type=pl.DeviceIdType.MESH,
                        ).wait_send()

        for s in range(N_DEV):
            @pl.when(my != s)
            def _(s=s):
                pltpu.make_async_remote_copy(
                    src_ref=pbuf.at[s], dst_ref=pbuf.at[s],
                    send_sem=ar_send_sems.at[s],
                    recv_sem=ar_recv_sems.at[s],
                    device_id=(s,),
                    device_id_type=pl.DeviceIdType.MESH,
                ).wait_recv()

        out_ref[...] = pbuf[0] + pbuf[1] + pbuf[2] + pbuf[3]

        for s in range(N_DEV):
            @pl.when(my == s)
            def _(s=s):
                for d in range(N_DEV):
                    if d == s:
                        continue
                    pltpu.make_async_remote_copy(
                        src_ref=pbuf.at[s], dst_ref=pbuf.at[s],
                        send_sem=ar_send_sems.at[d],
                        recv_sem=ar_recv_sems.at[s],
                        device_id=(d,),
                        device_id_type=pl.DeviceIdType.MESH,
                    ).wait_send()

    return pl.pallas_call(
        body,
        out_shape=jax.ShapeDtypeStruct((B, SQ, D_MODEL), jnp.float32),
        in_specs=[
            pl.BlockSpec(memory_space=pltpu.VMEM),
            pl.BlockSpec(memory_space=pltpu.VMEM),
            pl.BlockSpec(memory_space=pltpu.MemorySpace.HBM),
            pl.BlockSpec(memory_space=pltpu.MemorySpace.HBM),
            pl.BlockSpec(memory_space=pltpu.VMEM),
        ],
        out_specs=pl.BlockSpec(memory_space=pltpu.VMEM),
        scratch_shapes=[
            pltpu.VMEM((KV_SRC, B, SKV_LOC, H_LOC, DH), jnp.float32),
            pltpu.VMEM((KV_SRC, B, SKV_LOC, H_LOC, DH), jnp.float32),
            pltpu.VMEM((N_DEV, B, SQ, D_MODEL), jnp.float32),
            pltpu.SemaphoreType.DMA((N_DEV, 2)),
            pltpu.SemaphoreType.DMA((KV_SRC, 2)),
            pltpu.SemaphoreType.DMA((2,)),
            pltpu.SemaphoreType.DMA((N_DEV,)),
            pltpu.SemaphoreType.DMA((N_DEV,)),
        ],
        compiler_params=pltpu.CompilerParams(collective_id=0),
    )(x, Wq, K_ext, V_ext, Wo)


# device time: 44736 ns/iter; 1.0727x vs baseline; 1.0727x over previous
import jax
import jax.numpy as jnp
from jax import lax
from jax.experimental import pallas as pl
from jax.experimental.pallas import tpu as pltpu

N_DEV = 4
B, SQ, SKV, HQ, DH = 2, 128, 512, 16, 64
H_LOC = HQ // N_DEV
KV_SRC = 2
SKV_LOC = SKV // N_DEV
SKV_EFF = KV_SRC * SKV_LOC
D_MODEL = 512
WINDOW = 128
NEG = -1e9


def kernel(x, Wq, K_ext, V_ext, Wo):
    def body(x_ref, wq_ref, k_ref, v_ref, wo_ref, out_ref,
             kbuf, vbuf, pbuf,
             kv_send_sems, kv_recv_sems, loc_sems,
             ar_send_sems, ar_recv_sems):
        my = lax.axis_index("i")

        barrier = pltpu.get_barrier_semaphore()
        for off in range(1, N_DEV):
            pl.semaphore_signal(
                barrier, inc=1,
                device_id=((my + off) % N_DEV,),
                device_id_type=pl.DeviceIdType.MESH,
            )
        pl.semaphore_wait(barrier, N_DEV - 1)

        for s in range(KV_SRC):
            @pl.when(my == s)
            def _(s=s):
                for d in range(N_DEV):
                    if d == s:
                        continue
                    for t, (src, dst) in enumerate(((k_ref, kbuf), (v_ref, vbuf))):
                        pltpu.make_async_remote_copy(
                            src_ref=src.at[:, :, pl.ds(d * H_LOC, H_LOC), :],
                            dst_ref=dst.at[s],
                            send_sem=kv_send_sems.at[d, t],
                            recv_sem=kv_recv_sems.at[s, t],
                            device_id=(d,),
                            device_id_type=pl.DeviceIdType.MESH,
                        ).start()
                for t, (src, dst) in enumerate(((k_ref, kbuf), (v_ref, vbuf))):
                    pltpu.make_async_copy(
                        src.at[:, :, pl.ds(s * H_LOC, H_LOC), :],
                        dst.at[s], loc_sems.at[t],
                    ).start()

        q_all = [
            [jnp.dot(x_ref[b], wq_ref[:, h * DH:(h + 1) * DH],
                     preferred_element_type=jnp.float32)
             for h in range(H_LOC)]
            for b in range(B)
        ]

        for s in range(KV_SRC):
            @pl.when(my == s)
            def _(s=s):
                for t, (src, dst) in enumerate(((k_ref, kbuf), (v_ref, vbuf))):
                    pltpu.make_async_copy(
                        src.at[:, :, pl.ds(s * H_LOC, H_LOC), :],
                        dst.at[s], loc_sems.at[t],
                    ).wait()

        for s in range(KV_SRC):
            @pl.when(my != s)
            def _(s=s):
                for t, dst in ((0, kbuf), (1, vbuf)):
                    pltpu.make_async_remote_copy(
                        src_ref=k_ref.at[:, :, pl.ds(0, H_LOC), :],
                        dst_ref=dst.at[s],
                        send_sem=kv_send_sems.at[0, t],
                        recv_sem=kv_recv_sems.at[s, t],
                        device_id=(s,),
                        device_id_type=pl.DeviceIdType.MESH,
                    ).wait_recv()

        qi = lax.broadcasted_iota(jnp.int32, (SQ, SKV_EFF), 0)
        ki = lax.broadcasted_iota(jnp.int32, (SQ, SKV_EFF), 1)
        mask = ki <= qi + WINDOW

        for b in range(B):
            acc = jnp.zeros((SQ, D_MODEL), jnp.float32)
            for h in range(H_LOC):
                kc = kbuf[:, b, :, h, :].reshape(SKV_EFF, DH)
                vc = vbuf[:, b, :, h, :].reshape(SKV_EFF, DH)
                scores = lax.dot_general(
                    q_all[b][h], kc, (((1,), (1,)), ((), ())),
                    preferred_element_type=jnp.float32) * 0.125
                scores = jnp.where(mask, scores, NEG)
                m = jnp.max(scores, axis=1, keepdims=True)
                w = jnp.exp(scores - m)
                w = w / jnp.sum(w, axis=1, keepdims=True)
                ctx = jnp.dot(w, vc, preferred_element_type=jnp.float32)
                acc = acc + jnp.dot(ctx, wo_ref[h * DH:(h + 1) * DH, :],
                                    preferred_element_type=jnp.float32)
            for d in range(N_DEV):
                @pl.when(my == d)
                def _(d=d, b=b, acc=acc):
                    pbuf[d, b] = acc
                    for peer in range(N_DEV):
                        if peer == d:
                            continue
                        pltpu.make_async_remote_copy(
                            src_ref=pbuf.at[d, b],
                            dst_ref=pbuf.at[d, b],
                            send_sem=ar_send_sems.at[peer, b],
                            recv_sem=ar_recv_sems.at[d, b],
                            device_id=(peer,),
                            device_id_type=pl.DeviceIdType.MESH,
                        ).start()

        for s in range(KV_SRC):
            @pl.when(my == s)
            def _(s=s):
                for d in range(N_DEV):
                    if d == s:
                        continue
                    for t, (src, dst) in enumerate(((k_ref, kbuf), (v_ref, vbuf))):
                        pltpu.make_async_remote_copy(
                            src_ref=src.at[:, :, pl.ds(d * H_LOC, H_LOC), :],
                            dst_ref=dst.at[s],
                            send_sem=kv_send_sems.at[d, t],
                            recv_sem=kv_recv_sems.at[s, t],
                            device_id=(d,),
                            device_id_type=pl.DeviceIdType.MESH,
                        ).wait_send()

        for b in range(B):
            for s in range(N_DEV):
                @pl.when(my != s)
                def _(s=s, b=b):
                    pltpu.make_async_remote_copy(
                        src_ref=pbuf.at[s, b], dst_ref=pbuf.at[s, b],
                        send_sem=ar_send_sems.at[s, b],
                        recv_sem=ar_recv_sems.at[s, b],
                        device_id=(s,),
                        device_id_type=pl.DeviceIdType.MESH,
                    ).wait_recv()
            out_ref[b] = pbuf[0, b] + pbuf[1, b] + pbuf[2, b] + pbuf[3, b]

        for s in range(N_DEV):
            @pl.when(my == s)
            def _(s=s):
                for b in range(B):
                    for peer in range(N_DEV):
                        if peer == s:
                            continue
                        pltpu.make_async_remote_copy(
                            src_ref=pbuf.at[s, b], dst_ref=pbuf.at[s, b],
                            send_sem=ar_send_sems.at[peer, b],
                            recv_sem=ar_recv_sems.at[s, b],
                            device_id=(peer,),
                            device_id_type=pl.DeviceIdType.MESH,
                        ).wait_send()

    return pl.pallas_call(
        body,
        out_shape=jax.ShapeDtypeStruct((B, SQ, D_MODEL), jnp.float32),
        in_specs=[
            pl.BlockSpec(memory_space=pltpu.VMEM),
            pl.BlockSpec(memory_space=pltpu.VMEM),
            pl.BlockSpec(memory_space=pltpu.MemorySpace.HBM),
            pl.BlockSpec(memory_space=pltpu.MemorySpace.HBM),
            pl.BlockSpec(memory_space=pltpu.VMEM),
        ],
        out_specs=pl.BlockSpec(memory_space=pltpu.VMEM),
        scratch_shapes=[
            pltpu.VMEM((KV_SRC, B, SKV_LOC, H_LOC, DH), jnp.float32),
            pltpu.VMEM((KV_SRC, B, SKV_LOC, H_LOC, DH), jnp.float32),
            pltpu.VMEM((N_DEV, B, SQ, D_MODEL), jnp.float32),
            pltpu.SemaphoreType.DMA((N_DEV, 2)),
            pltpu.SemaphoreType.DMA((KV_SRC, 2)),
            pltpu.SemaphoreType.DMA((2,)),
            pltpu.SemaphoreType.DMA((N_DEV, B)),
            pltpu.SemaphoreType.DMA((N_DEV, B)),
        ],
        compiler_params=pltpu.CompilerParams(collective_id=0),
    )(x, Wq, K_ext, V_ext, Wo)
